# baseline (speedup 1.0000x reference)
import jax
import jax.numpy as jnp
from jax.experimental import pallas as pl
from jax.experimental.pallas import tpu as pltpu

_ROWS = 25600
_COLS = 1024
_N_CHUNKS = 8


def _memcpy_kernel(x_ref, o_ref, sems):
    chunk = _ROWS // _N_CHUNKS
    for i in range(_N_CHUNKS):
        pltpu.make_async_copy(
            x_ref.at[pl.ds(i * chunk, chunk), :],
            o_ref.at[pl.ds(i * chunk, chunk), :],
            sems.at[i],
        ).start()
    for i in range(_N_CHUNKS):
        pltpu.make_async_copy(
            x_ref.at[pl.ds(i * chunk, chunk), :],
            o_ref.at[pl.ds(i * chunk, chunk), :],
            sems.at[i],
        ).wait()


def kernel(x):
    n, d = x.shape
    flat = x.reshape(_ROWS, _COLS)
    out = pl.pallas_call(
        _memcpy_kernel,
        in_specs=[pl.BlockSpec(memory_space=pl.ANY)],
        out_specs=pl.BlockSpec(memory_space=pl.ANY),
        out_shape=jax.ShapeDtypeStruct(flat.shape, flat.dtype),
        scratch_shapes=[pltpu.SemaphoreType.DMA((_N_CHUNKS,))],
    )(flat)
    return out.reshape(n, d)


# speedup vs baseline: 5.6406x; 5.6406x over previous
import jax
import jax.numpy as jnp
from jax.experimental import pallas as pl

_ROWS = 25600
_COLS = 1024
_BLOCK_ROWS = 1024


def _copy_block(x_ref, o_ref):
    o_ref[...] = x_ref[...]


def kernel(x):
    n, d = x.shape
    flat = x.reshape(_ROWS, _COLS)
    out = pl.pallas_call(
        _copy_block,
        grid=(_ROWS // _BLOCK_ROWS,),
        in_specs=[pl.BlockSpec((_BLOCK_ROWS, _COLS), lambda i: (i, 0))],
        out_specs=pl.BlockSpec((_BLOCK_ROWS, _COLS), lambda i: (i, 0)),
        out_shape=jax.ShapeDtypeStruct(flat.shape, flat.dtype),
    )(flat)
    return out.reshape(n, d)


# speedup vs baseline: 12.4870x; 2.2138x over previous
import jax
import jax.numpy as jnp
from jax.experimental import pallas as pl

_BLOCK_ROWS = 8192


def _copy_block(x_ref, o_ref):
    o_ref[...] = x_ref[...]


def kernel(x):
    n, d = x.shape
    return pl.pallas_call(
        _copy_block,
        grid=(n // _BLOCK_ROWS,),
        in_specs=[pl.BlockSpec((_BLOCK_ROWS, d), lambda i: (i, 0))],
        out_specs=pl.BlockSpec((_BLOCK_ROWS, d), lambda i: (i, 0)),
        out_shape=jax.ShapeDtypeStruct(x.shape, x.dtype),
    )(x)


# speedup vs baseline: 12.5193x; 1.0026x over previous
import jax
import jax.numpy as jnp
from jax.experimental import pallas as pl
from jax.experimental.pallas import tpu as pltpu

_BLOCK_ROWS = 8192


def _copy_block(x_ref, o_ref):
    o_ref[...] = x_ref[...]


def kernel(x):
    n, d = x.shape
    return pl.pallas_call(
        _copy_block,
        grid=(n // _BLOCK_ROWS,),
        in_specs=[pl.BlockSpec((_BLOCK_ROWS, d), lambda i: (i, 0))],
        out_specs=pl.BlockSpec((_BLOCK_ROWS, d), lambda i: (i, 0)),
        out_shape=jax.ShapeDtypeStruct(x.shape, x.dtype),
        compiler_params=pltpu.CompilerParams(
            dimension_semantics=("parallel",),
        ),
    )(x)


# speedup vs baseline: 12.5676x; 1.0039x over previous
import jax
import jax.numpy as jnp
from jax.experimental import pallas as pl
from jax.experimental.pallas import tpu as pltpu

_BM = 4096
_DEPTH = 8
_LOOKAHEAD = _DEPTH // 2


def _copy_kernel(x_hbm, o_hbm, buf, sin, sout):
    nrows = x_hbm.shape[0]
    nblocks = nrows // _BM

    def cp_in(i):
        return pltpu.make_async_copy(
            x_hbm.at[pl.ds(i * _BM, _BM), :], buf.at[i % _DEPTH],
            sin.at[i % _DEPTH])

    def cp_out(i):
        return pltpu.make_async_copy(
            buf.at[i % _DEPTH], o_hbm.at[pl.ds(i * _BM, _BM), :],
            sout.at[i % _DEPTH])

    for i in range(min(_LOOKAHEAD, nblocks)):
        cp_in(i).start()
    for i in range(nblocks):
        cp_in(i).wait()
        cp_out(i).start()
        j = i + _LOOKAHEAD
        if j < nblocks:
            k = j - _DEPTH
            if k >= 0:
                cp_out(k).wait()
            cp_in(j).start()
    for k in range(max(0, nblocks - 2 * _LOOKAHEAD), nblocks):
        cp_out(k).wait()


def kernel(x):
    n, d = x.shape
    return pl.pallas_call(
        _copy_kernel,
        in_specs=[pl.BlockSpec(memory_space=pl.ANY)],
        out_specs=pl.BlockSpec(memory_space=pl.ANY),
        out_shape=jax.ShapeDtypeStruct(x.shape, x.dtype),
        scratch_shapes=[
            pltpu.VMEM((_DEPTH, _BM, d), jnp.float32),
            pltpu.SemaphoreType.DMA((_DEPTH,)),
            pltpu.SemaphoreType.DMA((_DEPTH,)),
        ],
    )(x)
